# Initial kernel scaffold; baseline (speedup 1.0000x reference)
import jax
import jax.numpy as jnp
from jax.experimental import pallas as pl


def kernel(inputs_shard, weight0, weight1, scatter_index, splits):
    raise NotImplementedError("write your pallas kernel here")


# speedup vs baseline: 3.7457x; 3.7457x over previous
import functools

import jax
import jax.numpy as jnp
from jax import lax
from jax.experimental import pallas as pl
from jax.experimental.pallas import tpu as pltpu
from jax.experimental.pallas import tpu_sc as plsc

NTOKENS = 2048
H = 1024
FFN = 4096
E = 8
TOPK = 2
TOTAL = NTOKENS * TOPK

BM = 512
NB = TOTAL // BM
G = NB + E - 1
BF = 1024
KF = FFN // BF

_NC = 2
_NS = 16
NW = _NC * _NS
PAIRS_W = TOTAL // NW
TOK_W = NTOKENS // NW


def _gelu(x):
    return 0.5 * x * (1.0 + lax.erf(x / jnp.sqrt(2.0).astype(x.dtype)))


def _sc_scatter_body(x_hbm, sia_hbm, sib_hbm, out_hbm, idx_a, idx_b, rows, sem):
    wid = lax.axis_index("s") * _NC + lax.axis_index("c")
    for c in range(TOK_W // 32):
        tb = wid * TOK_W + c * 32
        pltpu.sync_copy(sia_hbm.at[pl.ds(tb, 32)], idx_a)
        pltpu.sync_copy(sib_hbm.at[pl.ds(tb, 32)], idx_b)
        pltpu.sync_copy(x_hbm.at[pl.ds(tb, 32)], rows)
        pltpu.async_copy(rows, out_hbm.at[idx_a], sem).wait()
        pltpu.async_copy(rows, out_hbm.at[idx_b], sem).wait()


def _sc_scatter(inputs_shard, si_a, si_b):
    mesh = plsc.VectorSubcoreMesh(core_axis_name="c", subcore_axis_name="s")
    fn = pl.kernel(
        _sc_scatter_body,
        out_type=jax.ShapeDtypeStruct((TOTAL, H), jnp.float32),
        mesh=mesh,
        scratch_types=[
            pltpu.VMEM((32,), jnp.int32),
            pltpu.VMEM((32,), jnp.int32),
            pltpu.VMEM((32, H), jnp.float32),
            pltpu.SemaphoreType.DMA,
        ],
    )
    return fn(inputs_shard, si_a, si_b)


def _make_group_meta(splits):
    off = jnp.concatenate(
        [jnp.zeros((1,), jnp.int32), jnp.cumsum(splits).astype(jnp.int32)])
    lo_e = off[:-1]
    hi_e = off[1:]
    nonempty = splits > 0
    first_b = lo_e // BM
    last_b = (hi_e - 1) // BM
    count = jnp.where(nonempty, last_b - first_b + 1, 0)
    starts = jnp.concatenate(
        [jnp.zeros((1,), jnp.int32), jnp.cumsum(count).astype(jnp.int32)])
    g = jnp.arange(G, dtype=jnp.int32)
    eid = jnp.searchsorted(starts[1:], g, side="right").astype(jnp.int32)
    eid = jnp.clip(eid, 0, E - 1)
    valid = g < starts[E]
    bid = first_b[eid] + (g - starts[eid])
    bid = jnp.where(valid, bid, NB - 1)
    lo = jnp.where(valid, lo_e[eid], 0)
    hi = jnp.where(valid, hi_e[eid], 0)
    return jnp.stack([bid, eid, lo, hi]).astype(jnp.int32)


def _gemm_body(meta_ref, x_ref, w0_ref, w1_ref, out_ref):
    g = pl.program_id(0)
    k = pl.program_id(1)
    bid = meta_ref[0, g]
    lo = meta_ref[2, g]
    hi = meta_ref[3, g]
    gm1 = jnp.maximum(g - 1, 0)
    first_visit = jnp.logical_or(g == 0, meta_ref[0, gm1] != bid)
    init = jnp.logical_and(first_visit, k == 0)

    rows = bid * BM + lax.broadcasted_iota(jnp.int32, (BM, 1), 0)
    mask = jnp.logical_and(rows >= lo, rows < hi)
    x = jnp.where(mask, x_ref[...], 0.0)
    inter = _gelu(jnp.dot(x, w0_ref[0], preferred_element_type=jnp.float32))
    contrib = jnp.dot(inter, w1_ref[0], preferred_element_type=jnp.float32)

    @pl.when(init)
    def _():
        out_ref[...] = contrib

    @pl.when(jnp.logical_not(init))
    def _():
        out_ref[...] += contrib


def _grouped_gemm(meta, scattered, weight0, weight1):
    grid_spec = pltpu.PrefetchScalarGridSpec(
        num_scalar_prefetch=1,
        grid=(G, KF),
        in_specs=[
            pl.BlockSpec((BM, H), lambda g, k, m: (m[0, g], 0)),
            pl.BlockSpec((1, H, BF), lambda g, k, m: (m[1, g], 0, k)),
            pl.BlockSpec((1, BF, H), lambda g, k, m: (m[1, g], k, 0)),
        ],
        out_specs=pl.BlockSpec((BM, H), lambda g, k, m: (m[0, g], 0)),
    )
    return pl.pallas_call(
        _gemm_body,
        grid_spec=grid_spec,
        out_shape=jax.ShapeDtypeStruct((TOTAL, H), jnp.float32),
    )(meta, scattered, weight0, weight1)


def _sc_gather_body(y_hbm, sia_hbm, sib_hbm, out_a_hbm, out_b_hbm,
                    idx_a, idx_b, rows_a, rows_b, sem):
    wid = lax.axis_index("s") * _NC + lax.axis_index("c")
    for c in range(TOK_W // 32):
        tb = wid * TOK_W + c * 32
        pltpu.sync_copy(sia_hbm.at[pl.ds(tb, 32)], idx_a)
        pltpu.sync_copy(sib_hbm.at[pl.ds(tb, 32)], idx_b)
        ca = pltpu.async_copy(y_hbm.at[idx_a], rows_a, sem)
        cb = pltpu.async_copy(y_hbm.at[idx_b], rows_b, sem)
        ca.wait()
        cb.wait()
        pltpu.sync_copy(rows_a, out_a_hbm.at[pl.ds(tb, 32)])
        pltpu.sync_copy(rows_b, out_b_hbm.at[pl.ds(tb, 32)])


def _sc_gather(y, si_a, si_b):
    mesh = plsc.VectorSubcoreMesh(core_axis_name="c", subcore_axis_name="s")
    fn = pl.kernel(
        _sc_gather_body,
        out_type=(
            jax.ShapeDtypeStruct((NTOKENS, H), jnp.float32),
            jax.ShapeDtypeStruct((NTOKENS, H), jnp.float32),
        ),
        mesh=mesh,
        scratch_types=[
            pltpu.VMEM((32,), jnp.int32),
            pltpu.VMEM((32,), jnp.int32),
            pltpu.VMEM((32, H), jnp.float32),
            pltpu.VMEM((32, H), jnp.float32),
            pltpu.SemaphoreType.DMA,
        ],
    )
    return fn(y, si_a, si_b)


def _sum_body(a_ref, b_ref, o_ref):
    o_ref[...] = a_ref[...] + b_ref[...]


def _pair_sum(a, b):
    blk = 256
    return pl.pallas_call(
        _sum_body,
        grid=(NTOKENS // blk,),
        in_specs=[
            pl.BlockSpec((blk, H), lambda i: (i, 0)),
            pl.BlockSpec((blk, H), lambda i: (i, 0)),
        ],
        out_specs=pl.BlockSpec((blk, H), lambda i: (i, 0)),
        out_shape=jax.ShapeDtypeStruct((NTOKENS, H), jnp.float32),
    )(a, b)


def kernel(inputs_shard, weight0, weight1, scatter_index, splits):
    si_a = scatter_index[:, 0].astype(jnp.int32)
    si_b = scatter_index[:, 1].astype(jnp.int32)
    meta = _make_group_meta(splits)
    scattered = _sc_scatter(inputs_shard, si_a, si_b)
    y = _grouped_gemm(meta, scattered, weight0, weight1)
    out_a, out_b = _sc_gather(y, si_a, si_b)
    return _pair_sum(out_a, out_b)


# speedup vs baseline: 3.7462x; 1.0001x over previous
import functools

import jax
import jax.numpy as jnp
from jax import lax
from jax.experimental import pallas as pl
from jax.experimental.pallas import tpu as pltpu
from jax.experimental.pallas import tpu_sc as plsc

NTOKENS = 2048
H = 1024
FFN = 4096
E = 8
TOPK = 2
TOTAL = NTOKENS * TOPK

BM = 512
NB = TOTAL // BM
G = NB + E - 1
BF = 1024
KF = FFN // BF

_NC = 2
_NS = 16
NW = _NC * _NS
PAIRS_W = TOTAL // NW
TOK_W = NTOKENS // NW


def _gelu(x):
    return 0.5 * x * (1.0 + lax.erf(x / jnp.sqrt(2.0).astype(x.dtype)))


def _sc_scatter_body(x_hbm, sia_hbm, sib_hbm, out_hbm, idx_a, idx_b, rows, sem):
    wid = lax.axis_index("s") * _NC + lax.axis_index("c")
    for c in range(TOK_W // 32):
        tb = wid * TOK_W + c * 32
        pltpu.sync_copy(sia_hbm.at[pl.ds(tb, 32)], idx_a)
        pltpu.sync_copy(sib_hbm.at[pl.ds(tb, 32)], idx_b)
        pltpu.sync_copy(x_hbm.at[pl.ds(tb, 32)], rows)
        pltpu.async_copy(rows, out_hbm.at[idx_a], sem).wait()
        pltpu.async_copy(rows, out_hbm.at[idx_b], sem).wait()


def _sc_scatter(inputs_shard, si_a, si_b):
    mesh = plsc.VectorSubcoreMesh(core_axis_name="c", subcore_axis_name="s")
    fn = pl.kernel(
        _sc_scatter_body,
        out_type=jax.ShapeDtypeStruct((TOTAL, H), jnp.float32),
        mesh=mesh,
        scratch_types=[
            pltpu.VMEM((32,), jnp.int32),
            pltpu.VMEM((32,), jnp.int32),
            pltpu.VMEM((32, H), jnp.float32),
            pltpu.SemaphoreType.DMA,
        ],
    )
    return fn(inputs_shard, si_a, si_b)


def _make_group_meta(splits):
    off = jnp.concatenate(
        [jnp.zeros((1,), jnp.int32), jnp.cumsum(splits).astype(jnp.int32)])
    lo_e = off[:-1]
    hi_e = off[1:]
    nonempty = splits > 0
    first_b = lo_e // BM
    last_b = (hi_e - 1) // BM
    count = jnp.where(nonempty, last_b - first_b + 1, 0)
    starts = jnp.concatenate(
        [jnp.zeros((1,), jnp.int32), jnp.cumsum(count).astype(jnp.int32)])
    g = jnp.arange(G, dtype=jnp.int32)
    eid = jnp.searchsorted(starts[1:], g, side="right").astype(jnp.int32)
    eid = jnp.clip(eid, 0, E - 1)
    valid = g < starts[E]
    bid = first_b[eid] + (g - starts[eid])
    bid = jnp.where(valid, bid, NB - 1)
    lo = jnp.where(valid, lo_e[eid], 0)
    hi = jnp.where(valid, hi_e[eid], 0)
    return jnp.stack([bid, eid, lo, hi]).astype(jnp.int32)


def _gemm_body(meta_ref, x_ref, w0_ref, w1_ref, out_ref):
    g = pl.program_id(0)
    k = pl.program_id(1)
    bid = meta_ref[0, g]
    lo = meta_ref[2, g]
    hi = meta_ref[3, g]
    gm1 = jnp.maximum(g - 1, 0)
    first_visit = jnp.logical_or(g == 0, meta_ref[0, gm1] != bid)
    init = jnp.logical_and(first_visit, k == 0)

    rows = bid * BM + lax.broadcasted_iota(jnp.int32, (BM, 1), 0)
    mask = jnp.logical_and(rows >= lo, rows < hi)
    x = jnp.where(mask, x_ref[...], 0.0).astype(jnp.bfloat16)
    inter = _gelu(jnp.dot(x, w0_ref[0].astype(jnp.bfloat16),
                          preferred_element_type=jnp.float32))
    contrib = jnp.dot(inter.astype(jnp.bfloat16),
                      w1_ref[0].astype(jnp.bfloat16),
                      preferred_element_type=jnp.float32)

    @pl.when(init)
    def _():
        out_ref[...] = contrib

    @pl.when(jnp.logical_not(init))
    def _():
        out_ref[...] += contrib


def _grouped_gemm(meta, scattered, weight0, weight1):
    grid_spec = pltpu.PrefetchScalarGridSpec(
        num_scalar_prefetch=1,
        grid=(G, KF),
        in_specs=[
            pl.BlockSpec((BM, H), lambda g, k, m: (m[0, g], 0)),
            pl.BlockSpec((1, H, BF), lambda g, k, m: (m[1, g], 0, k)),
            pl.BlockSpec((1, BF, H), lambda g, k, m: (m[1, g], k, 0)),
        ],
        out_specs=pl.BlockSpec((BM, H), lambda g, k, m: (m[0, g], 0)),
    )
    return pl.pallas_call(
        _gemm_body,
        grid_spec=grid_spec,
        out_shape=jax.ShapeDtypeStruct((TOTAL, H), jnp.float32),
    )(meta, scattered, weight0, weight1)


def _sc_gather_body(y_hbm, sia_hbm, sib_hbm, out_a_hbm, out_b_hbm,
                    idx_a, idx_b, rows_a, rows_b, sem):
    wid = lax.axis_index("s") * _NC + lax.axis_index("c")
    for c in range(TOK_W // 32):
        tb = wid * TOK_W + c * 32
        pltpu.sync_copy(sia_hbm.at[pl.ds(tb, 32)], idx_a)
        pltpu.sync_copy(sib_hbm.at[pl.ds(tb, 32)], idx_b)
        ca = pltpu.async_copy(y_hbm.at[idx_a], rows_a, sem)
        cb = pltpu.async_copy(y_hbm.at[idx_b], rows_b, sem)
        ca.wait()
        cb.wait()
        pltpu.sync_copy(rows_a, out_a_hbm.at[pl.ds(tb, 32)])
        pltpu.sync_copy(rows_b, out_b_hbm.at[pl.ds(tb, 32)])


def _sc_gather(y, si_a, si_b):
    mesh = plsc.VectorSubcoreMesh(core_axis_name="c", subcore_axis_name="s")
    fn = pl.kernel(
        _sc_gather_body,
        out_type=(
            jax.ShapeDtypeStruct((NTOKENS, H), jnp.float32),
            jax.ShapeDtypeStruct((NTOKENS, H), jnp.float32),
        ),
        mesh=mesh,
        scratch_types=[
            pltpu.VMEM((32,), jnp.int32),
            pltpu.VMEM((32,), jnp.int32),
            pltpu.VMEM((32, H), jnp.float32),
            pltpu.VMEM((32, H), jnp.float32),
            pltpu.SemaphoreType.DMA,
        ],
    )
    return fn(y, si_a, si_b)


def _sum_body(a_ref, b_ref, o_ref):
    o_ref[...] = a_ref[...] + b_ref[...]


def _pair_sum(a, b):
    blk = 256
    return pl.pallas_call(
        _sum_body,
        grid=(NTOKENS // blk,),
        in_specs=[
            pl.BlockSpec((blk, H), lambda i: (i, 0)),
            pl.BlockSpec((blk, H), lambda i: (i, 0)),
        ],
        out_specs=pl.BlockSpec((blk, H), lambda i: (i, 0)),
        out_shape=jax.ShapeDtypeStruct((NTOKENS, H), jnp.float32),
    )(a, b)


def kernel(inputs_shard, weight0, weight1, scatter_index, splits):
    si_a = scatter_index[:, 0].astype(jnp.int32)
    si_b = scatter_index[:, 1].astype(jnp.int32)
    meta = _make_group_meta(splits)
    scattered = _sc_scatter(inputs_shard, si_a, si_b)
    y = _grouped_gemm(meta, scattered, weight0, weight1)
    out_a, out_b = _sc_gather(y, si_a, si_b)
    return _pair_sum(out_a, out_b)


# speedup vs baseline: 4.3276x; 1.1552x over previous
import functools

import jax
import jax.numpy as jnp
from jax import lax
from jax.experimental import pallas as pl
from jax.experimental.pallas import tpu as pltpu
from jax.experimental.pallas import tpu_sc as plsc

NTOKENS = 2048
H = 1024
FFN = 4096
E = 8
TOPK = 2
TOTAL = NTOKENS * TOPK

BM = 256
NB = TOTAL // BM
BF = 1024
KF = FFN // BF

_NC = 2
_NS = 16
NW = _NC * _NS
PAIRS_W = TOTAL // NW
TOK_W = NTOKENS // NW


def _gelu(x):
    return 0.5 * x * (1.0 + lax.erf(x / jnp.sqrt(2.0).astype(x.dtype)))


def _sc_scatter_body(x_hbm, sia_hbm, sib_hbm, out_hbm, idx_a, idx_b, rows, sem):
    wid = lax.axis_index("s") * _NC + lax.axis_index("c")
    for c in range(TOK_W // 32):
        tb = wid * TOK_W + c * 32
        pltpu.sync_copy(sia_hbm.at[pl.ds(tb, 32)], idx_a)
        pltpu.sync_copy(sib_hbm.at[pl.ds(tb, 32)], idx_b)
        pltpu.sync_copy(x_hbm.at[pl.ds(tb, 32)], rows)
        pltpu.async_copy(rows, out_hbm.at[idx_a], sem).wait()
        pltpu.async_copy(rows, out_hbm.at[idx_b], sem).wait()


def _sc_scatter(inputs_shard, si_a, si_b):
    mesh = plsc.VectorSubcoreMesh(core_axis_name="c", subcore_axis_name="s")
    fn = pl.kernel(
        _sc_scatter_body,
        out_type=jax.ShapeDtypeStruct((TOTAL, H), jnp.float32),
        mesh=mesh,
        scratch_types=[
            pltpu.VMEM((32,), jnp.int32),
            pltpu.VMEM((32,), jnp.int32),
            pltpu.VMEM((32, H), jnp.float32),
            pltpu.SemaphoreType.DMA,
        ],
    )
    return fn(inputs_shard, si_a, si_b)


def _make_group_meta(splits):
    return jnp.concatenate(
        [jnp.zeros((1,), jnp.int32), jnp.cumsum(splits).astype(jnp.int32)])


def _gemm_body(off_ref, x_ref, w0_ref, w1_ref, out_ref):
    e = pl.program_id(0)
    k = pl.program_id(1)
    lo = off_ref[e]
    hi = off_ref[e + 1]

    @pl.when(jnp.logical_and(e == 0, k == 0))
    def _():
        out_ref[...] = jnp.zeros_like(out_ref)

    first = lo // BM
    nb = jnp.where(hi > lo, (hi - 1) // BM - first + 1, 0)
    w0b = w0_ref[0].astype(jnp.bfloat16)
    w1b = w1_ref[0].astype(jnp.bfloat16)

    def body(i, carry):
        b = first + i
        rows = b * BM + lax.broadcasted_iota(jnp.int32, (BM, 1), 0)
        mask = jnp.logical_and(rows >= lo, rows < hi)
        xb = x_ref[pl.ds(b * BM, BM), :]
        xb = jnp.where(mask, xb, 0.0).astype(jnp.bfloat16)
        inter = _gelu(jnp.dot(xb, w0b, preferred_element_type=jnp.float32))
        contrib = jnp.dot(inter.astype(jnp.bfloat16), w1b,
                          preferred_element_type=jnp.float32)
        out_ref[pl.ds(b * BM, BM), :] += contrib
        return carry

    lax.fori_loop(0, nb, body, 0)


def _grouped_gemm(off, scattered, weight0, weight1):
    grid_spec = pltpu.PrefetchScalarGridSpec(
        num_scalar_prefetch=1,
        grid=(E, KF),
        in_specs=[
            pl.BlockSpec((TOTAL, H), lambda e, k, m: (0, 0)),
            pl.BlockSpec((1, H, BF), lambda e, k, m: (e, 0, k)),
            pl.BlockSpec((1, BF, H), lambda e, k, m: (e, k, 0)),
        ],
        out_specs=pl.BlockSpec((TOTAL, H), lambda e, k, m: (0, 0)),
    )
    return pl.pallas_call(
        _gemm_body,
        grid_spec=grid_spec,
        out_shape=jax.ShapeDtypeStruct((TOTAL, H), jnp.float32),
    )(off, scattered, weight0, weight1)


def _sc_gather_body(y_hbm, sia_hbm, sib_hbm, out_a_hbm, out_b_hbm,
                    idx_a, idx_b, rows_a, rows_b, sem):
    wid = lax.axis_index("s") * _NC + lax.axis_index("c")
    for c in range(TOK_W // 32):
        tb = wid * TOK_W + c * 32
        pltpu.sync_copy(sia_hbm.at[pl.ds(tb, 32)], idx_a)
        pltpu.sync_copy(sib_hbm.at[pl.ds(tb, 32)], idx_b)
        ca = pltpu.async_copy(y_hbm.at[idx_a], rows_a, sem)
        cb = pltpu.async_copy(y_hbm.at[idx_b], rows_b, sem)
        ca.wait()
        cb.wait()
        pltpu.sync_copy(rows_a, out_a_hbm.at[pl.ds(tb, 32)])
        pltpu.sync_copy(rows_b, out_b_hbm.at[pl.ds(tb, 32)])


def _sc_gather(y, si_a, si_b):
    mesh = plsc.VectorSubcoreMesh(core_axis_name="c", subcore_axis_name="s")
    fn = pl.kernel(
        _sc_gather_body,
        out_type=(
            jax.ShapeDtypeStruct((NTOKENS, H), jnp.float32),
            jax.ShapeDtypeStruct((NTOKENS, H), jnp.float32),
        ),
        mesh=mesh,
        scratch_types=[
            pltpu.VMEM((32,), jnp.int32),
            pltpu.VMEM((32,), jnp.int32),
            pltpu.VMEM((32, H), jnp.float32),
            pltpu.VMEM((32, H), jnp.float32),
            pltpu.SemaphoreType.DMA,
        ],
    )
    return fn(y, si_a, si_b)


def _sum_body(a_ref, b_ref, o_ref):
    o_ref[...] = a_ref[...] + b_ref[...]


def _pair_sum(a, b):
    blk = 256
    return pl.pallas_call(
        _sum_body,
        grid=(NTOKENS // blk,),
        in_specs=[
            pl.BlockSpec((blk, H), lambda i: (i, 0)),
            pl.BlockSpec((blk, H), lambda i: (i, 0)),
        ],
        out_specs=pl.BlockSpec((blk, H), lambda i: (i, 0)),
        out_shape=jax.ShapeDtypeStruct((NTOKENS, H), jnp.float32),
    )(a, b)


def kernel(inputs_shard, weight0, weight1, scatter_index, splits):
    si_a = scatter_index[:, 0].astype(jnp.int32)
    si_b = scatter_index[:, 1].astype(jnp.int32)
    meta = _make_group_meta(splits)
    scattered = _sc_scatter(inputs_shard, si_a, si_b)
    y = _grouped_gemm(meta, scattered, weight0, weight1)
    out_a, out_b = _sc_gather(y, si_a, si_b)
    return _pair_sum(out_a, out_b)


# speedup vs baseline: 4.3315x; 1.0009x over previous
import jax
import jax.numpy as jnp
from jax import lax
from jax.experimental import pallas as pl
from jax.experimental.pallas import tpu as pltpu
from jax.experimental.pallas import tpu_sc as plsc

NTOKENS = 2048
H = 1024
FFN = 4096
E = 8
TOPK = 2
TOTAL = NTOKENS * TOPK

BM = 256
BF = 1024
KF = FFN // BF

_NC = 2
_NS = 16
NW = _NC * _NS
TOK_W = NTOKENS // NW


def _gelu(x):
    return 0.5 * x * (1.0 + lax.erf(x / jnp.sqrt(2.0).astype(x.dtype)))


def _sc_scatter_body(x_hbm, sia_hbm, sib_hbm, out_hbm, idx_a, idx_b, rows, sem):
    wid = lax.axis_index("s") * _NC + lax.axis_index("c")
    for c in range(TOK_W // 32):
        tb = wid * TOK_W + c * 32
        pltpu.sync_copy(sia_hbm.at[pl.ds(tb, 32)], idx_a)
        pltpu.sync_copy(sib_hbm.at[pl.ds(tb, 32)], idx_b)
        pltpu.sync_copy(x_hbm.at[pl.ds(tb, 32)], rows)
        pltpu.async_copy(rows, out_hbm.at[idx_a], sem).wait()
        pltpu.async_copy(rows, out_hbm.at[idx_b], sem).wait()


def _sc_scatter(inputs_shard, si_a, si_b):
    mesh = plsc.VectorSubcoreMesh(core_axis_name="c", subcore_axis_name="s")
    fn = pl.kernel(
        _sc_scatter_body,
        out_type=jax.ShapeDtypeStruct((TOTAL, H), jnp.float32),
        mesh=mesh,
        scratch_types=[
            pltpu.VMEM((32,), jnp.int32),
            pltpu.VMEM((32,), jnp.int32),
            pltpu.VMEM((32, H), jnp.float32),
            pltpu.SemaphoreType.DMA,
        ],
    )
    return fn(inputs_shard, si_a, si_b)


def _make_group_meta(splits):
    return jnp.concatenate(
        [jnp.zeros((1,), jnp.int32), jnp.cumsum(splits).astype(jnp.int32)])


def _gemm_body(off_ref, x_ref, w0_ref, w1_ref, out_ref):
    e = pl.program_id(0)
    k = pl.program_id(1)
    lo = off_ref[e]
    hi = off_ref[e + 1]

    @pl.when(jnp.logical_and(e == 0, k == 0))
    def _():
        out_ref[...] = jnp.zeros_like(out_ref)

    first = lo // BM
    nb = jnp.where(hi > lo, (hi - 1) // BM - first + 1, 0)
    w0b = w0_ref[0].astype(jnp.bfloat16)
    w1b = w1_ref[0].astype(jnp.bfloat16)

    def body(i, carry):
        b = first + i
        rows = b * BM + lax.broadcasted_iota(jnp.int32, (BM, 1), 0)
        mask = jnp.logical_and(rows >= lo, rows < hi)
        xb = x_ref[pl.ds(b * BM, BM), :]
        xb = jnp.where(mask, xb, 0.0).astype(jnp.bfloat16)
        inter = _gelu(jnp.dot(xb, w0b, preferred_element_type=jnp.float32))
        contrib = jnp.dot(inter.astype(jnp.bfloat16), w1b,
                          preferred_element_type=jnp.float32)
        out_ref[pl.ds(b * BM, BM), :] += contrib
        return carry

    lax.fori_loop(0, nb, body, 0)


def _grouped_gemm(off, scattered, weight0, weight1):
    grid_spec = pltpu.PrefetchScalarGridSpec(
        num_scalar_prefetch=1,
        grid=(E, KF),
        in_specs=[
            pl.BlockSpec((TOTAL, H), lambda e, k, m: (0, 0)),
            pl.BlockSpec((1, H, BF), lambda e, k, m: (e, 0, k)),
            pl.BlockSpec((1, BF, H), lambda e, k, m: (e, k, 0)),
        ],
        out_specs=pl.BlockSpec((TOTAL, H), lambda e, k, m: (0, 0)),
    )
    return pl.pallas_call(
        _gemm_body,
        grid_spec=grid_spec,
        out_shape=jax.ShapeDtypeStruct((TOTAL, H), jnp.float32),
    )(off, scattered, weight0, weight1)


def _sc_gather_body(y_hbm, sia_hbm, sib_hbm, out_a_hbm, out_b_hbm,
                    idx_a, idx_b, rows_a, rows_b, sem):
    wid = lax.axis_index("s") * _NC + lax.axis_index("c")
    for c in range(TOK_W // 32):
        tb = wid * TOK_W + c * 32
        pltpu.sync_copy(sia_hbm.at[pl.ds(tb, 32)], idx_a)
        pltpu.sync_copy(sib_hbm.at[pl.ds(tb, 32)], idx_b)
        ca = pltpu.async_copy(y_hbm.at[idx_a], rows_a, sem)
        cb = pltpu.async_copy(y_hbm.at[idx_b], rows_b, sem)
        ca.wait()
        cb.wait()
        pltpu.sync_copy(rows_a, out_a_hbm.at[pl.ds(tb, 32)])
        pltpu.sync_copy(rows_b, out_b_hbm.at[pl.ds(tb, 32)])


def _sc_gather(y, si_a, si_b):
    mesh = plsc.VectorSubcoreMesh(core_axis_name="c", subcore_axis_name="s")
    fn = pl.kernel(
        _sc_gather_body,
        out_type=(
            jax.ShapeDtypeStruct((NTOKENS, H), jnp.float32),
            jax.ShapeDtypeStruct((NTOKENS, H), jnp.float32),
        ),
        mesh=mesh,
        scratch_types=[
            pltpu.VMEM((32,), jnp.int32),
            pltpu.VMEM((32,), jnp.int32),
            pltpu.VMEM((32, H), jnp.float32),
            pltpu.VMEM((32, H), jnp.float32),
            pltpu.SemaphoreType.DMA,
        ],
    )
    return fn(y, si_a, si_b)


def _sum_body(a_ref, b_ref, o_ref):
    o_ref[...] = a_ref[...] + b_ref[...]


def _pair_sum(a, b):
    blk = 256
    return pl.pallas_call(
        _sum_body,
        grid=(NTOKENS // blk,),
        in_specs=[
            pl.BlockSpec((blk, H), lambda i: (i, 0)),
            pl.BlockSpec((blk, H), lambda i: (i, 0)),
        ],
        out_specs=pl.BlockSpec((blk, H), lambda i: (i, 0)),
        out_shape=jax.ShapeDtypeStruct((NTOKENS, H), jnp.float32),
    )(a, b)


def kernel(inputs_shard, weight0, weight1, scatter_index, splits):
    si_a = scatter_index[:, 0].astype(jnp.int32)
    si_b = scatter_index[:, 1].astype(jnp.int32)
    off = _make_group_meta(splits)
    scattered = _sc_scatter(inputs_shard, si_a, si_b)
    y = _grouped_gemm(off, scattered, weight0, weight1)
    out_a, out_b = _sc_gather(y, si_a, si_b)
    return _pair_sum(out_a, out_b)
